# baseline (speedup 1.0000x reference)
import functools

import jax
import jax.numpy as jnp
from jax import lax
from jax.experimental import pallas as pl
from jax.experimental.pallas import tpu as pltpu
from jax.experimental.pallas import tpu_sc as plsc

B = 16
TOTAL_UIH = 32768
NUM_CAND = 128
DIM = 128
TOTAL_OUT = TOTAL_UIH + B * NUM_CAND

NC, NS, L = 2, 16, 16
NW = NC * NS
PER_W = TOTAL_OUT // NW
CHUNKS = [(g * 128, 128) for g in range(8)] + [(1024, 64)]
NBUF = 4
UIH_SLICE = 3200


def _build_emb_gather():
    mesh = plsc.VectorSubcoreMesh(
        core_axis_name="c", subcore_axis_name="s",
        num_cores=NC, num_subcores=NS)

    @functools.partial(
        pl.kernel,
        out_type=jax.ShapeDtypeStruct((TOTAL_OUT, DIM), jnp.float32),
        mesh=mesh,
        compiler_params=pltpu.CompilerParams(needs_layout_passes=False),
        scratch_types=[
            pltpu.VMEM((UIH_SLICE,), jnp.int32),
            pltpu.VMEM((B * NUM_CAND,), jnp.int32),
            pltpu.VMEM((128,), jnp.int32),
            pltpu.VMEM((128,), jnp.int32),
            pltpu.VMEM((128,), jnp.int32),
            pltpu.VMEM((PER_W,), jnp.int32),
            [pltpu.VMEM((128, DIM), jnp.float32) for _ in range(NBUF)],
            [pltpu.SemaphoreType.DMA for _ in range(NBUF)],
            [pltpu.SemaphoreType.DMA for _ in range(NBUF)],
            pltpu.SemaphoreType.DMA,
        ],
    )
    def emb_gather(table_hbm, uih_hbm, cand_hbm, inner_hbm, out_hbm,
                   uih_v, cand_v, inner_v, u_v, t_v, idx_v,
                   bufs, gsems, wsems, ssem):
        wid = lax.axis_index("s") * NC + lax.axis_index("c")
        base = wid * PER_W
        start_w = pl.multiple_of(
            jnp.clip(base - B * NUM_CAND, 0, TOTAL_UIH - UIH_SLICE), 8)

        stage_uih = pltpu.async_copy(
            uih_hbm.at[pl.ds(start_w, UIH_SLICE)], uih_v, ssem)
        pltpu.sync_copy(cand_hbm, cand_v)
        pltpu.sync_copy(inner_hbm, inner_v.at[pl.ds(0, B - 1)])

        iota = lax.iota(jnp.int32, L)
        inner16 = plsc.load_gather(inner_v, [jnp.clip(iota, 0, B - 2)])
        u_vec = jnp.where(iota >= B - 1, TOTAL_UIH, inner16)
        t_vec = u_vec + (iota + 1) * NUM_CAND
        u_v[pl.ds(0, L)] = u_vec
        t_v[pl.ds(B, L)] = t_vec
        thr = [plsc.load_gather(t_v, [jnp.full((L,), B + k, jnp.int32)])
               for k in range(B)]
        stage_uih.wait()

        def compute_ids(v, carry):
            p = (base + v * L) + iota
            s = jnp.zeros((L,), jnp.int32)
            for k in range(B):
                s = s + (p >= thr[k]).astype(jnp.int32)
            unext = plsc.load_gather(u_v, [s])
            is_cand = p >= unext + s * NUM_CAND
            uidx = jnp.clip(p - s * NUM_CAND - start_w, 0, UIH_SLICE - 1)
            cidx = jnp.clip(p - unext, 0, B * NUM_CAND - 1)
            uid = plsc.load_gather(uih_v, [uidx])
            cid = plsc.load_gather(cand_v, [cidx])
            idx_v[pl.ds(v * L, L)] = jnp.where(is_cand, cid, uid)
            return carry

        gcp = [None] * len(CHUNKS)
        wcp = [None] * len(CHUNKS)
        for g, (off, sz) in enumerate(CHUNKS):
            bi = g % NBUF
            lax.fori_loop(off // L, (off + sz) // L, compute_ids, 0)
            if g >= NBUF:
                wcp[g - NBUF].wait()
            gcp[g] = pltpu.async_copy(
                table_hbm.at[idx_v.at[pl.ds(off, sz)]],
                bufs[bi].at[pl.ds(0, sz)], gsems[bi])
            if g >= 1:
                poff, psz = CHUNKS[g - 1]
                gcp[g - 1].wait()
                wcp[g - 1] = pltpu.async_copy(
                    bufs[(g - 1) % NBUF].at[pl.ds(0, psz)],
                    out_hbm.at[pl.ds(base + poff, psz)],
                    wsems[(g - 1) % NBUF])
        last = len(CHUNKS) - 1
        loff, lsz = CHUNKS[last]
        gcp[last].wait()
        wcp[last] = pltpu.async_copy(
            bufs[last % NBUF].at[pl.ds(0, lsz)],
            out_hbm.at[pl.ds(base + loff, lsz)], wsems[last % NBUF])
        for g in range(max(0, len(CHUNKS) - NBUF), len(CHUNKS)):
            wcp[g].wait()

    return emb_gather


_emb_gather = _build_emb_gather()


def kernel(uih_values, uih_inner_offsets, cand_values, uih_timestamps, table):
    b = uih_inner_offsets.shape[0] + 1
    total_uih = uih_values.shape[0]
    nc = cand_values.shape[0] // b

    seq_emb_values = _emb_gather(
        table, uih_values, cand_values, uih_inner_offsets.astype(jnp.int32))

    u_arr = jnp.concatenate([
        uih_inner_offsets.astype(jnp.int32),
        jnp.array([total_uih], dtype=jnp.int32),
    ])
    uih_seq_lengths = u_arr - jnp.concatenate(
        [jnp.zeros((1,), jnp.int32), u_arr[:-1]])
    seq_emb_lengths = uih_seq_lengths + nc
    num_candidates = jnp.full((b,), nc, dtype=jnp.int32)

    return (seq_emb_values, seq_emb_lengths, uih_timestamps,
            uih_seq_lengths, num_candidates)


# speedup vs baseline: 1.0305x; 1.0305x over previous
import functools

import jax
import jax.numpy as jnp
from jax import lax
from jax.experimental import pallas as pl
from jax.experimental.pallas import tpu as pltpu
from jax.experimental.pallas import tpu_sc as plsc

B = 16
TOTAL_UIH = 32768
NUM_CAND = 128
DIM = 128
TOTAL_OUT = TOTAL_UIH + B * NUM_CAND

NC, NS, L = 2, 16, 16
NW = NC * NS
PER_W = TOTAL_OUT // NW
CHUNKS = [(g * 128, 128) for g in range(8)] + [(1024, 64)]
NBUF = 4
UIH_SLICE = 3200


def _build_emb_gather():
    mesh = plsc.VectorSubcoreMesh(
        core_axis_name="c", subcore_axis_name="s",
        num_cores=NC, num_subcores=NS)

    @functools.partial(
        pl.kernel,
        out_type=jax.ShapeDtypeStruct((TOTAL_OUT, DIM), jnp.float32),
        mesh=mesh,
        compiler_params=pltpu.CompilerParams(needs_layout_passes=False),
        scratch_types=[
            pltpu.VMEM((UIH_SLICE,), jnp.int32),
            pltpu.VMEM((B * NUM_CAND,), jnp.int32),
            pltpu.VMEM((128,), jnp.int32),
            pltpu.VMEM((128,), jnp.int32),
            pltpu.VMEM((128,), jnp.int32),
            pltpu.VMEM((PER_W,), jnp.int32),
            [pltpu.VMEM((128, DIM), jnp.float32) for _ in range(NBUF)],
            [pltpu.SemaphoreType.DMA for _ in range(NBUF)],
            [pltpu.SemaphoreType.DMA for _ in range(NBUF)],
            pltpu.SemaphoreType.DMA,
        ],
    )
    def emb_gather(table_hbm, uih_hbm, cand_hbm, inner_hbm, out_hbm,
                   uih_v, cand_v, inner_v, u_v, t_v, idx_v,
                   bufs, gsems, wsems, ssem):
        wid = lax.axis_index("s") * NC + lax.axis_index("c")
        base = wid * PER_W
        start_w = pl.multiple_of(
            jnp.clip(base - B * NUM_CAND, 0, TOTAL_UIH - UIH_SLICE), 8)

        stage_uih = pltpu.async_copy(
            uih_hbm.at[pl.ds(start_w, UIH_SLICE)], uih_v, ssem)
        pltpu.sync_copy(cand_hbm, cand_v)
        pltpu.sync_copy(inner_hbm, inner_v.at[pl.ds(0, B - 1)])

        iota = lax.iota(jnp.int32, L)
        inner16 = plsc.load_gather(inner_v, [jnp.clip(iota, 0, B - 2)])
        u_vec = jnp.where(iota >= B - 1, TOTAL_UIH, inner16)
        t_vec = u_vec + (iota + 1) * NUM_CAND
        u_v[pl.ds(0, L)] = u_vec
        t_v[pl.ds(B, L)] = t_vec
        thr = [plsc.load_gather(t_v, [jnp.full((L,), B + k, jnp.int32)])
               for k in range(B)]
        stage_uih.wait()

        def compute_ids(v, carry):
            p = (base + v * L) + iota
            s = jnp.zeros((L,), jnp.int32)
            for k in range(B):
                s = s + (p >= thr[k]).astype(jnp.int32)
            unext = plsc.load_gather(u_v, [s])
            is_cand = p >= unext + s * NUM_CAND
            uidx = jnp.clip(p - s * NUM_CAND - start_w, 0, UIH_SLICE - 1)
            cidx = jnp.clip(p - unext, 0, B * NUM_CAND - 1)
            uid = plsc.load_gather(uih_v, [uidx])
            cid = plsc.load_gather(cand_v, [cidx])
            idx_v[pl.ds(v * L, L)] = jnp.where(is_cand, cid, uid)
            return carry

        gcp = [None] * len(CHUNKS)
        wcp = [None] * len(CHUNKS)
        for g in range(2):
            off, sz = CHUNKS[g]
            lax.fori_loop(off // L, (off + sz) // L, compute_ids, 0)
            gcp[g] = pltpu.async_copy(
                table_hbm.at[idx_v.at[pl.ds(off, sz)]],
                bufs[g].at[pl.ds(0, sz)], gsems[g])
        lax.fori_loop(CHUNKS[2][0] // L, PER_W // L, compute_ids, 0)

        gcp[0].wait()
        wcp[0] = pltpu.async_copy(
            bufs[0].at[pl.ds(0, CHUNKS[0][1])],
            out_hbm.at[pl.ds(base, CHUNKS[0][1])], wsems[0])

        for g, (off, sz) in enumerate(CHUNKS):
            if g < 2:
                continue
            bi = g % NBUF
            if g >= NBUF:
                wcp[g - NBUF].wait()
            gcp[g] = pltpu.async_copy(
                table_hbm.at[idx_v.at[pl.ds(off, sz)]],
                bufs[bi].at[pl.ds(0, sz)], gsems[bi])
            if g >= 1:
                poff, psz = CHUNKS[g - 1]
                gcp[g - 1].wait()
                wcp[g - 1] = pltpu.async_copy(
                    bufs[(g - 1) % NBUF].at[pl.ds(0, psz)],
                    out_hbm.at[pl.ds(base + poff, psz)],
                    wsems[(g - 1) % NBUF])
        last = len(CHUNKS) - 1
        loff, lsz = CHUNKS[last]
        gcp[last].wait()
        wcp[last] = pltpu.async_copy(
            bufs[last % NBUF].at[pl.ds(0, lsz)],
            out_hbm.at[pl.ds(base + loff, lsz)], wsems[last % NBUF])
        for g in range(max(0, len(CHUNKS) - NBUF), len(CHUNKS)):
            wcp[g].wait()

    return emb_gather


_emb_gather = _build_emb_gather()


def kernel(uih_values, uih_inner_offsets, cand_values, uih_timestamps, table):
    b = uih_inner_offsets.shape[0] + 1
    total_uih = uih_values.shape[0]
    nc = cand_values.shape[0] // b

    seq_emb_values = _emb_gather(
        table, uih_values, cand_values, uih_inner_offsets.astype(jnp.int32))

    u_arr = jnp.concatenate([
        uih_inner_offsets.astype(jnp.int32),
        jnp.array([total_uih], dtype=jnp.int32),
    ])
    uih_seq_lengths = u_arr - jnp.concatenate(
        [jnp.zeros((1,), jnp.int32), u_arr[:-1]])
    seq_emb_lengths = uih_seq_lengths + nc
    num_candidates = jnp.full((b,), nc, dtype=jnp.int32)

    return (seq_emb_values, seq_emb_lengths, uih_timestamps,
            uih_seq_lengths, num_candidates)


# speedup vs baseline: 1.0371x; 1.0065x over previous
import functools

import jax
import jax.numpy as jnp
from jax import lax
from jax.experimental import pallas as pl
from jax.experimental.pallas import tpu as pltpu
from jax.experimental.pallas import tpu_sc as plsc

B = 16
TOTAL_UIH = 32768
NUM_CAND = 128
DIM = 128
TOTAL_OUT = TOTAL_UIH + B * NUM_CAND

NC, NS, L = 2, 16, 16
NW = NC * NS
PER_W = TOTAL_OUT // NW
CHUNKS = [(g * 128, 128) for g in range(8)] + [(1024, 64)]
NBUF = 4
UIH_SLICE = 3200


def _build_emb_gather():
    mesh = plsc.VectorSubcoreMesh(
        core_axis_name="c", subcore_axis_name="s",
        num_cores=NC, num_subcores=NS)

    @functools.partial(
        pl.kernel,
        out_type=jax.ShapeDtypeStruct((TOTAL_OUT, DIM), jnp.float32),
        mesh=mesh,
        compiler_params=pltpu.CompilerParams(needs_layout_passes=False),
        scratch_types=[
            pltpu.VMEM((UIH_SLICE,), jnp.int32),
            pltpu.VMEM((B * NUM_CAND,), jnp.int32),
            pltpu.VMEM((128,), jnp.int32),
            pltpu.VMEM((128,), jnp.int32),
            pltpu.VMEM((128,), jnp.int32),
            pltpu.VMEM((PER_W,), jnp.int32),
            [pltpu.VMEM((128, DIM), jnp.float32) for _ in range(NBUF)],
            [pltpu.SemaphoreType.DMA for _ in range(NBUF)],
            [pltpu.SemaphoreType.DMA for _ in range(NBUF)],
            pltpu.SemaphoreType.DMA,
        ],
    )
    def emb_gather(table_hbm, uih_hbm, cand_hbm, inner_hbm, out_hbm,
                   uih_v, cand_v, inner_v, u_v, t_v, idx_v,
                   bufs, gsems, wsems, ssem):
        wid = lax.axis_index("s") * NC + lax.axis_index("c")
        base = wid * PER_W
        start_w = pl.multiple_of(
            jnp.clip(base - B * NUM_CAND, 0, TOTAL_UIH - UIH_SLICE), 8)

        stage_uih = pltpu.async_copy(
            uih_hbm.at[pl.ds(start_w, UIH_SLICE)], uih_v, ssem)
        pltpu.sync_copy(cand_hbm, cand_v)
        pltpu.sync_copy(inner_hbm, inner_v.at[pl.ds(0, B - 1)])

        iota = lax.iota(jnp.int32, L)
        inner16 = plsc.load_gather(inner_v, [jnp.clip(iota, 0, B - 2)])
        u_vec = jnp.where(iota >= B - 1, TOTAL_UIH, inner16)
        t_vec = u_vec + (iota + 1) * NUM_CAND
        u_v[pl.ds(0, L)] = u_vec
        t_v[pl.ds(B, L)] = t_vec
        thr = [plsc.load_gather(t_v, [jnp.full((L,), B + k, jnp.int32)])
               for k in range(B)]
        stage_uih.wait()

        def compute_ids(v, carry):
            p = (base + v * L) + iota
            s = jnp.zeros((L,), jnp.int32)
            for k in range(B):
                s = s + (p >= thr[k]).astype(jnp.int32)
            unext = plsc.load_gather(u_v, [s])
            is_cand = p >= unext + s * NUM_CAND
            uidx = jnp.clip(p - s * NUM_CAND - start_w, 0, UIH_SLICE - 1)
            cidx = jnp.clip(p - unext, 0, B * NUM_CAND - 1)
            uid = plsc.load_gather(uih_v, [uidx])
            cid = plsc.load_gather(cand_v, [cidx])
            idx_v[pl.ds(v * L, L)] = jnp.where(is_cand, cid, uid)
            return carry

        first_sz = CHUNKS[0][1]
        lax.fori_loop(0, first_sz // L, compute_ids, 0)
        gcp = [None] * len(CHUNKS)
        wcp = [None] * len(CHUNKS)
        gcp[0] = pltpu.async_copy(
            table_hbm.at[idx_v.at[pl.ds(0, first_sz)]],
            bufs[0].at[pl.ds(0, first_sz)], gsems[0])
        lax.fori_loop(first_sz // L, PER_W // L, compute_ids, 0)

        for g, (off, sz) in enumerate(CHUNKS):
            if g == 0:
                continue
            bi = g % NBUF
            if g >= NBUF:
                wcp[g - NBUF].wait()
            gcp[g] = pltpu.async_copy(
                table_hbm.at[idx_v.at[pl.ds(off, sz)]],
                bufs[bi].at[pl.ds(0, sz)], gsems[bi])
            if g >= 1:
                poff, psz = CHUNKS[g - 1]
                gcp[g - 1].wait()
                wcp[g - 1] = pltpu.async_copy(
                    bufs[(g - 1) % NBUF].at[pl.ds(0, psz)],
                    out_hbm.at[pl.ds(base + poff, psz)],
                    wsems[(g - 1) % NBUF])
        last = len(CHUNKS) - 1
        loff, lsz = CHUNKS[last]
        gcp[last].wait()
        wcp[last] = pltpu.async_copy(
            bufs[last % NBUF].at[pl.ds(0, lsz)],
            out_hbm.at[pl.ds(base + loff, lsz)], wsems[last % NBUF])
        for g in range(max(0, len(CHUNKS) - NBUF), len(CHUNKS)):
            wcp[g].wait()

    return emb_gather


_emb_gather = _build_emb_gather()


def kernel(uih_values, uih_inner_offsets, cand_values, uih_timestamps, table):
    b = uih_inner_offsets.shape[0] + 1
    total_uih = uih_values.shape[0]
    nc = cand_values.shape[0] // b

    seq_emb_values = _emb_gather(
        table, uih_values, cand_values, uih_inner_offsets.astype(jnp.int32))

    u_arr = jnp.concatenate([
        uih_inner_offsets.astype(jnp.int32),
        jnp.array([total_uih], dtype=jnp.int32),
    ])
    uih_seq_lengths = u_arr - jnp.concatenate(
        [jnp.zeros((1,), jnp.int32), u_arr[:-1]])
    seq_emb_lengths = uih_seq_lengths + nc
    num_candidates = jnp.full((b,), nc, dtype=jnp.int32)

    return (seq_emb_values, seq_emb_lengths, uih_timestamps,
            uih_seq_lengths, num_candidates)


# speedup vs baseline: 1.0499x; 1.0123x over previous
import functools

import jax
import jax.numpy as jnp
from jax import lax
from jax.experimental import pallas as pl
from jax.experimental.pallas import tpu as pltpu
from jax.experimental.pallas import tpu_sc as plsc

B = 16
TOTAL_UIH = 32768
NUM_CAND = 128
DIM = 128
TOTAL_OUT = TOTAL_UIH + B * NUM_CAND

NC, NS, L = 2, 16, 16
NW = NC * NS
PER_W = TOTAL_OUT // NW
CHUNKS = [(g * 128, 128) for g in range(8)] + [(1024, 64)]
NBUF = 4
UIH_SLICE = 3200


def _build_emb_gather():
    mesh = plsc.VectorSubcoreMesh(
        core_axis_name="c", subcore_axis_name="s",
        num_cores=NC, num_subcores=NS)

    @functools.partial(
        pl.kernel,
        out_type=jax.ShapeDtypeStruct((TOTAL_OUT, DIM), jnp.float32),
        mesh=mesh,
        compiler_params=pltpu.CompilerParams(needs_layout_passes=False),
        scratch_types=[
            pltpu.VMEM((UIH_SLICE,), jnp.int32),
            pltpu.VMEM((B * NUM_CAND,), jnp.int32),
            pltpu.VMEM((128,), jnp.int32),
            pltpu.VMEM((128,), jnp.int32),
            pltpu.VMEM((128,), jnp.int32),
            pltpu.VMEM((PER_W,), jnp.int32),
            [pltpu.VMEM((128, DIM), jnp.float32) for _ in range(NBUF)],
            [pltpu.SemaphoreType.DMA for _ in range(NBUF)],
            [pltpu.SemaphoreType.DMA for _ in range(NBUF)],
            pltpu.SemaphoreType.DMA,
            pltpu.SemaphoreType.DMA,
        ],
    )
    def emb_gather(table_hbm, uih_hbm, cand_hbm, inner_hbm, out_hbm,
                   uih_v, cand_v, inner_v, u_v, t_v, idx_v,
                   bufs, gsems, wsems, ssem, csem):
        wid = lax.axis_index("s") * NC + lax.axis_index("c")
        base = wid * PER_W
        start_w = pl.multiple_of(
            jnp.clip(base - B * NUM_CAND, 0, TOTAL_UIH - UIH_SLICE), 8)

        stage_uih = pltpu.async_copy(
            uih_hbm.at[pl.ds(start_w, UIH_SLICE)], uih_v, ssem)
        stage_cand = pltpu.async_copy(cand_hbm, cand_v, csem)
        pltpu.sync_copy(inner_hbm, inner_v.at[pl.ds(0, B - 1)])

        iota = lax.iota(jnp.int32, L)
        inner16 = plsc.load_gather(inner_v, [jnp.clip(iota, 0, B - 2)])
        u_vec = jnp.where(iota >= B - 1, TOTAL_UIH, inner16)
        t_vec = u_vec + (iota + 1) * NUM_CAND
        u_v[pl.ds(0, L)] = u_vec
        t_v[pl.ds(B, L)] = t_vec
        thr = [plsc.load_gather(t_v, [jnp.full((L,), B + k, jnp.int32)])
               for k in range(B)]
        stage_cand.wait()
        stage_uih.wait()

        def compute_ids(v, carry):
            p = (base + v * L) + iota
            s = jnp.zeros((L,), jnp.int32)
            for k in range(B):
                s = s + (p >= thr[k]).astype(jnp.int32)
            unext = plsc.load_gather(u_v, [s])
            is_cand = p >= unext + s * NUM_CAND
            uidx = jnp.minimum(p - s * NUM_CAND - start_w, UIH_SLICE - 1)
            cidx = jnp.maximum(p - unext, 0)
            uid = plsc.load_gather(uih_v, [uidx])
            cid = plsc.load_gather(cand_v, [cidx])
            idx_v[pl.ds(v * L, L)] = jnp.where(is_cand, cid, uid)
            return carry

        first_sz = CHUNKS[0][1]
        lax.fori_loop(0, first_sz // L, compute_ids, 0)
        gcp = [None] * len(CHUNKS)
        wcp = [None] * len(CHUNKS)
        gcp[0] = pltpu.async_copy(
            table_hbm.at[idx_v.at[pl.ds(0, first_sz)]],
            bufs[0].at[pl.ds(0, first_sz)], gsems[0])
        lax.fori_loop(first_sz // L, PER_W // L, compute_ids, 0)

        for g, (off, sz) in enumerate(CHUNKS):
            if g == 0:
                continue
            bi = g % NBUF
            if g >= NBUF:
                wcp[g - NBUF].wait()
            gcp[g] = pltpu.async_copy(
                table_hbm.at[idx_v.at[pl.ds(off, sz)]],
                bufs[bi].at[pl.ds(0, sz)], gsems[bi])
            if g >= 1:
                poff, psz = CHUNKS[g - 1]
                gcp[g - 1].wait()
                wcp[g - 1] = pltpu.async_copy(
                    bufs[(g - 1) % NBUF].at[pl.ds(0, psz)],
                    out_hbm.at[pl.ds(base + poff, psz)],
                    wsems[(g - 1) % NBUF])
        last = len(CHUNKS) - 1
        loff, lsz = CHUNKS[last]
        gcp[last].wait()
        wcp[last] = pltpu.async_copy(
            bufs[last % NBUF].at[pl.ds(0, lsz)],
            out_hbm.at[pl.ds(base + loff, lsz)], wsems[last % NBUF])
        for g in range(max(0, len(CHUNKS) - NBUF), len(CHUNKS)):
            wcp[g].wait()

    return emb_gather


_emb_gather = _build_emb_gather()


def kernel(uih_values, uih_inner_offsets, cand_values, uih_timestamps, table):
    b = uih_inner_offsets.shape[0] + 1
    total_uih = uih_values.shape[0]
    nc = cand_values.shape[0] // b

    seq_emb_values = _emb_gather(
        table, uih_values, cand_values, uih_inner_offsets.astype(jnp.int32))

    u_arr = jnp.concatenate([
        uih_inner_offsets.astype(jnp.int32),
        jnp.array([total_uih], dtype=jnp.int32),
    ])
    uih_seq_lengths = u_arr - jnp.concatenate(
        [jnp.zeros((1,), jnp.int32), u_arr[:-1]])
    seq_emb_lengths = uih_seq_lengths + nc
    num_candidates = jnp.full((b,), nc, dtype=jnp.int32)

    return (seq_emb_values, seq_emb_lengths, uih_timestamps,
            uih_seq_lengths, num_candidates)


# speedup vs baseline: 1.0625x; 1.0120x over previous
import functools

import jax
import jax.numpy as jnp
from jax import lax
from jax.experimental import pallas as pl
from jax.experimental.pallas import tpu as pltpu
from jax.experimental.pallas import tpu_sc as plsc

B = 16
TOTAL_UIH = 32768
NUM_CAND = 128
DIM = 128
TOTAL_OUT = TOTAL_UIH + B * NUM_CAND

NC, NS, L = 2, 16, 16
NW = NC * NS
PER_W = TOTAL_OUT // NW
CHUNKS = [(g * 128, 128) for g in range(8)] + [(1024, 64)]
NBUF = 4
UIH_SLICE = 3200


def _build_emb_gather():
    mesh = plsc.VectorSubcoreMesh(
        core_axis_name="c", subcore_axis_name="s",
        num_cores=NC, num_subcores=NS)

    @functools.partial(
        pl.kernel,
        out_type=(
            jax.ShapeDtypeStruct((TOTAL_OUT, DIM), jnp.float32),
            jax.ShapeDtypeStruct((B,), jnp.int32),
            jax.ShapeDtypeStruct((B,), jnp.int32),
            jax.ShapeDtypeStruct((B,), jnp.int32),
        ),
        mesh=mesh,
        compiler_params=pltpu.CompilerParams(needs_layout_passes=False),
        scratch_types=[
            pltpu.VMEM((UIH_SLICE,), jnp.int32),
            pltpu.VMEM((B * NUM_CAND,), jnp.int32),
            pltpu.VMEM((128,), jnp.int32),
            pltpu.VMEM((128,), jnp.int32),
            pltpu.VMEM((128,), jnp.int32),
            pltpu.VMEM((64,), jnp.int32),
            pltpu.VMEM((PER_W,), jnp.int32),
            [pltpu.VMEM((128, DIM), jnp.float32) for _ in range(NBUF)],
            [pltpu.SemaphoreType.DMA for _ in range(NBUF)],
            [pltpu.SemaphoreType.DMA for _ in range(NBUF)],
            pltpu.SemaphoreType.DMA,
            pltpu.SemaphoreType.DMA,
        ],
    )
    def emb_gather(table_hbm, uih_hbm, cand_hbm, inner_hbm,
                   out_hbm, slen_hbm, ulen_hbm, ncand_hbm,
                   uih_v, cand_v, inner_v, u_v, t_v, aux_v, idx_v,
                   bufs, gsems, wsems, ssem, csem):
        wid = lax.axis_index("s") * NC + lax.axis_index("c")
        base = wid * PER_W
        start_w = pl.multiple_of(
            jnp.clip(base - B * NUM_CAND, 0, TOTAL_UIH - UIH_SLICE), 8)

        stage_uih = pltpu.async_copy(
            uih_hbm.at[pl.ds(start_w, UIH_SLICE)], uih_v, ssem)
        stage_cand = pltpu.async_copy(cand_hbm, cand_v, csem)
        pltpu.sync_copy(inner_hbm, inner_v.at[pl.ds(0, B - 1)])

        iota = lax.iota(jnp.int32, L)
        inner16 = plsc.load_gather(inner_v, [jnp.clip(iota, 0, B - 2)])
        u_vec = jnp.where(iota >= B - 1, TOTAL_UIH, inner16)
        t_vec = u_vec + (iota + 1) * NUM_CAND
        u_v[pl.ds(0, L)] = u_vec
        t_v[pl.ds(B, L)] = t_vec
        thr = [plsc.load_gather(t_v, [jnp.full((L,), B + k, jnp.int32)])
               for k in range(B)]

        @pl.when(wid == 0)
        def _aux():
            aux_v[pl.ds(0, L)] = jnp.zeros((L,), jnp.int32)
            aux_v[pl.ds(1, L)] = u_vec
            prev = aux_v[pl.ds(0, L)]
            ulen = u_vec - prev
            aux_v[pl.ds(16, L)] = ulen
            pltpu.sync_copy(aux_v.at[pl.ds(16, L)], ulen_hbm)
            aux_v[pl.ds(32, L)] = ulen + NUM_CAND
            pltpu.sync_copy(aux_v.at[pl.ds(32, L)], slen_hbm)
            aux_v[pl.ds(48, L)] = jnp.full((L,), NUM_CAND, jnp.int32)
            pltpu.sync_copy(aux_v.at[pl.ds(48, L)], ncand_hbm)

        stage_cand.wait()
        stage_uih.wait()

        def compute_ids(v, carry):
            p = (base + v * L) + iota
            s = jnp.zeros((L,), jnp.int32)
            for k in range(B):
                s = s + (p >= thr[k]).astype(jnp.int32)
            unext = plsc.load_gather(u_v, [s])
            is_cand = p >= unext + s * NUM_CAND
            uidx = jnp.minimum(p - s * NUM_CAND - start_w, UIH_SLICE - 1)
            cidx = jnp.maximum(p - unext, 0)
            uid = plsc.load_gather(uih_v, [uidx])
            cid = plsc.load_gather(cand_v, [cidx])
            idx_v[pl.ds(v * L, L)] = jnp.where(is_cand, cid, uid)
            return carry

        first_sz = CHUNKS[0][1]
        lax.fori_loop(0, first_sz // L, compute_ids, 0)
        gcp = [None] * len(CHUNKS)
        wcp = [None] * len(CHUNKS)
        gcp[0] = pltpu.async_copy(
            table_hbm.at[idx_v.at[pl.ds(0, first_sz)]],
            bufs[0].at[pl.ds(0, first_sz)], gsems[0])
        lax.fori_loop(first_sz // L, PER_W // L, compute_ids, 0)

        for g, (off, sz) in enumerate(CHUNKS):
            if g == 0:
                continue
            bi = g % NBUF
            if g >= NBUF:
                wcp[g - NBUF].wait()
            gcp[g] = pltpu.async_copy(
                table_hbm.at[idx_v.at[pl.ds(off, sz)]],
                bufs[bi].at[pl.ds(0, sz)], gsems[bi])
            if g >= 1:
                poff, psz = CHUNKS[g - 1]
                gcp[g - 1].wait()
                wcp[g - 1] = pltpu.async_copy(
                    bufs[(g - 1) % NBUF].at[pl.ds(0, psz)],
                    out_hbm.at[pl.ds(base + poff, psz)],
                    wsems[(g - 1) % NBUF])
        last = len(CHUNKS) - 1
        loff, lsz = CHUNKS[last]
        gcp[last].wait()
        wcp[last] = pltpu.async_copy(
            bufs[last % NBUF].at[pl.ds(0, lsz)],
            out_hbm.at[pl.ds(base + loff, lsz)], wsems[last % NBUF])
        for g in range(max(0, len(CHUNKS) - NBUF), len(CHUNKS)):
            wcp[g].wait()

    return emb_gather


_emb_gather = _build_emb_gather()


def kernel(uih_values, uih_inner_offsets, cand_values, uih_timestamps, table):
    b = uih_inner_offsets.shape[0] + 1
    total_uih = uih_values.shape[0]
    nc = cand_values.shape[0] // b

    seq_emb_values, seq_emb_lengths, uih_seq_lengths, num_candidates = (
        _emb_gather(table, uih_values, cand_values,
                    uih_inner_offsets.astype(jnp.int32)))

    return (seq_emb_values, seq_emb_lengths, uih_timestamps,
            uih_seq_lengths, num_candidates)
